# baseline (speedup 1.0000x reference)
import jax
import jax.numpy as jnp
from jax.experimental import pallas as pl
from jax.experimental.pallas import tpu as pltpu

_BM = 512


def _mm_kernel(m_ref, x_ref, o_ref):
    o_ref[...] = jnp.dot(
        m_ref[...].astype(jnp.bfloat16),
        x_ref[...],
        preferred_element_type=jnp.float32,
    )


def kernel(inp, matrix):
    B, C, S = inp.shape
    M, K = matrix.shape
    x = inp.reshape(B * C, S).astype(jnp.bfloat16)
    out = pl.pallas_call(
        _mm_kernel,
        grid=(M // _BM,),
        in_specs=[
            pl.BlockSpec((_BM, K), lambda i: (i, 0)),
            pl.BlockSpec((K, S), lambda i: (0, 0)),
        ],
        out_specs=pl.BlockSpec((_BM, S), lambda i: (i, 0)),
        out_shape=jax.ShapeDtypeStruct((M, S), jnp.float32),
        compiler_params=pltpu.CompilerParams(
            dimension_semantics=("parallel",),
        ),
    )(matrix, x)
    return out.reshape(B, C, S)


# speedup vs baseline: 1.1267x; 1.1267x over previous
import jax
import jax.numpy as jnp
from jax.experimental import pallas as pl
from jax.experimental.pallas import tpu as pltpu

_BM = 512


def _mm_kernel(m_ref, x_ref, o_ref, xb_ref):
    @pl.when(pl.program_id(0) == 0)
    def _():
        xb_ref[...] = x_ref[...].astype(jnp.bfloat16)

    o_ref[...] = jnp.dot(
        m_ref[...].astype(jnp.bfloat16),
        xb_ref[...],
        preferred_element_type=jnp.float32,
    )


def kernel(inp, matrix):
    B, C, S = inp.shape
    M, K = matrix.shape
    x = inp.reshape(B * C, S)
    out = pl.pallas_call(
        _mm_kernel,
        grid=(M // _BM,),
        in_specs=[
            pl.BlockSpec((_BM, K), lambda i: (i, 0)),
            pl.BlockSpec((K, S), lambda i: (0, 0)),
        ],
        out_specs=pl.BlockSpec((_BM, S), lambda i: (i, 0)),
        out_shape=jax.ShapeDtypeStruct((M, S), jnp.float32),
        scratch_shapes=[pltpu.VMEM((K, S), jnp.bfloat16)],
        compiler_params=pltpu.CompilerParams(
            dimension_semantics=("arbitrary",),
        ),
    )(matrix, x)
    return out.reshape(B, C, S)
